# Initial kernel scaffold; baseline (speedup 1.0000x reference)
import functools

import jax
import jax.numpy as jnp
from jax import lax
from jax.experimental import pallas as pl
from jax.experimental.pallas import tpu as pltpu
from jax.experimental.pallas import tpu_sc as plsc

N = 100000
K = 16
C_IN = 16
C_OUT = 64
C_MID = 16

NK = N * K
NUM_WORKERS = 32
PER_W = NK // NUM_WORKERS
CHUNK = 2000
NCHUNK = PER_W // CHUNK

B = 1000
GRID = N // B


def _sc_gather(feats, xyz4, idx_flat):
    mesh = plsc.VectorSubcoreMesh(core_axis_name="c", subcore_axis_name="s")

    @functools.partial(
        pl.kernel,
        mesh=mesh,
        out_type=[
            jax.ShapeDtypeStruct((NK, C_IN), jnp.float32),
            jax.ShapeDtypeStruct((NK, 4), jnp.float32),
        ],
        scratch_types=[
            pltpu.VMEM((CHUNK,), jnp.int32),
            pltpu.VMEM((CHUNK, C_IN), jnp.float32),
            pltpu.VMEM((CHUNK, 4), jnp.float32),
            pltpu.SemaphoreType.DMA,
            pltpu.SemaphoreType.DMA,
        ],
    )
    def gather_kernel(feats_hbm, xyz_hbm, idx_hbm, gfeat_hbm, gxyz_hbm,
                      idx_v, frows_v, xrows_v, fsem, xsem):
        wid = lax.axis_index("s") * 2 + lax.axis_index("c")
        base = pl.multiple_of(wid * PER_W, 8)

        def body(j, carry):
            off = pl.multiple_of(base + j * CHUNK, 8)
            pltpu.sync_copy(idx_hbm.at[pl.ds(off, CHUNK)], idx_v)
            fcp = pltpu.async_copy(feats_hbm.at[idx_v], frows_v, fsem)
            xcp = pltpu.async_copy(xyz_hbm.at[idx_v], xrows_v, xsem)
            fcp.wait()
            pltpu.sync_copy(frows_v, gfeat_hbm.at[pl.ds(off, CHUNK)])
            xcp.wait()
            pltpu.sync_copy(xrows_v, gxyz_hbm.at[pl.ds(off, CHUNK)])
            return carry

        lax.fori_loop(0, NCHUNK, body, 0)

    return gather_kernel(feats, xyz4, idx_flat)


def _tc_body(xyz_ref, gx_ref, gf_ref,
             d_ref, t_ref, w1_ref, b1_ref, w2_ref, b2_ref, w3_ref, b3_ref,
             e_ref, f_ref, lwt_ref, lb_ref,
             local_ref, out_ref):
    x = xyz_ref[...]
    gx4 = gx_ref[...]
    f32 = jnp.float32
    local = (jnp.dot(gx4, d_ref[...], preferred_element_type=f32)
             - jnp.dot(x, t_ref[...], preferred_element_type=f32))
    local_ref[...] = local

    h1 = jnp.maximum(jnp.dot(local, w1_ref[...], preferred_element_type=f32)
                     + b1_ref[...], 0.0)
    h2 = jnp.dot(h1, w2_ref[...], preferred_element_type=f32) + b2_ref[...]
    w = jnp.maximum(jnp.dot(h2, w3_ref[...], preferred_element_type=f32)
                    + b3_ref[...], 0.0)

    gf = gf_ref[...]
    e = e_ref[...]
    f = f_ref[...]
    pre = jnp.zeros((B, C_IN * C_MID), f32)
    for k in range(K):
        gfk = gf[:, k * C_IN:(k + 1) * C_IN]
        wk = w[:, k * C_MID:(k + 1) * C_MID]
        pre = pre + (jnp.dot(gfk, e, preferred_element_type=f32)
                     * jnp.dot(wk, f, preferred_element_type=f32))

    out = jnp.dot(pre, lwt_ref[...], preferred_element_type=f32) + lb_ref[...]
    out_ref[...] = jnp.maximum(out, 0.0)


def _tc_dense(xyz, gx, gf, d, t, w1b, b1b, w2b, b2b, w3b, b3b, e, f, lwt, lb2):
    full = lambda shape: pl.BlockSpec(shape, lambda i: (0, 0))
    row = lambda width: pl.BlockSpec((B, width), lambda i: (i, 0))
    return pl.pallas_call(
        _tc_body,
        grid=(GRID,),
        in_specs=[
            row(3), row(4 * K), row(C_IN * K),
            full((4 * K, 3 * K)), full((3, 3 * K)),
            full((3 * K, 8 * K)), full((1, 8 * K)),
            full((8 * K, 8 * K)), full((1, 8 * K)),
            full((8 * K, C_MID * K)), full((1, C_MID * K)),
            full((C_IN, C_IN * C_MID)), full((C_MID, C_IN * C_MID)),
            full((C_IN * C_MID, C_OUT)), full((1, C_OUT)),
        ],
        out_specs=[
            pl.BlockSpec((B, 3 * K), lambda i: (i, 0)),
            pl.BlockSpec((B, C_OUT), lambda i: (i, 0)),
        ],
        out_shape=[
            jax.ShapeDtypeStruct((N, 3 * K), jnp.float32),
            jax.ShapeDtypeStruct((N, C_OUT), jnp.float32),
        ],
    )(xyz, gx, gf, d, t, w1b, b1b, w2b, b2b, w3b, b3b, e, f, lwt, lb2)


def kernel(dense_xyz, dense_feats, nei_inds,
           w1, b1, g1, be1, w2, b2, g2, be2, w3, b3, g3, be3, lw, lb):
    xyz = dense_xyz[0]
    feats = dense_feats[0]
    idx_flat = nei_inds[0].reshape(-1).astype(jnp.int32)
    xyz4 = jnp.pad(xyz, ((0, 0), (0, 1)))

    gfeat, gxyz4 = _sc_gather(feats, xyz4, idx_flat)
    gf2 = gfeat.reshape(N, K * C_IN)
    gx2 = gxyz4.reshape(N, K * 4)

    inv = 1.0 / jnp.sqrt(1.0 + 1e-5)
    s1, s2, s3 = g1 * inv, g2 * inv, g3 * inv
    w1e = w1.T * s1[None, :]
    c1 = b1 * s1 + be1
    w2e = w2.T * s2[None, :]
    c2 = b2 * s2 + be2
    w3e = w3.T * s3[None, :]
    c3 = b3 * s3 + be3

    eyeK = jnp.eye(K, dtype=jnp.float32)
    w1b = jnp.kron(eyeK, w1e)
    w2b = jnp.kron(eyeK, w2e)
    w3b = jnp.kron(eyeK, w3e)
    b1b = jnp.tile(c1, K)[None, :]
    b2b = jnp.tile(c2, K)[None, :]
    b3b = jnp.tile(c3, K)[None, :]

    d = jnp.kron(eyeK, jnp.eye(4, 3, dtype=jnp.float32))
    t = jnp.tile(jnp.eye(3, dtype=jnp.float32), (1, K))
    e = jnp.kron(jnp.eye(C_IN, dtype=jnp.float32),
                 jnp.ones((1, C_MID), jnp.float32))
    f = jnp.tile(jnp.eye(C_MID, dtype=jnp.float32), (1, C_IN))

    local48, new_feat = _tc_dense(xyz, gx2, gf2, d, t,
                                  w1b, b1b, w2b, b2b, w3b, b3b,
                                  e, f, lw.T, lb[None, :])
    return (new_feat[None], local48.reshape(1, N, K, 3))


# speedup vs baseline: 6.0333x; 6.0333x over previous
import functools

import jax
import jax.numpy as jnp
from jax import lax
from jax.experimental import pallas as pl
from jax.experimental.pallas import tpu as pltpu
from jax.experimental.pallas import tpu_sc as plsc

N = 100000
K = 16
C_IN = 16
C_OUT = 64
C_MID = 16
ROW = 32

NK = N * K
NUM_WORKERS = 32
PER_W = NK // NUM_WORKERS
CHUNK = 1000
NCHUNK = PER_W // CHUNK

B = 1000
GRID = N // B


def _sc_gather(table, idx_flat):
    mesh = plsc.VectorSubcoreMesh(core_axis_name="c", subcore_axis_name="s")

    @functools.partial(
        pl.kernel,
        mesh=mesh,
        out_type=jax.ShapeDtypeStruct((NK, ROW), jnp.float32),
        scratch_types=[
            pltpu.VMEM((CHUNK,), jnp.int32),
            pltpu.VMEM((CHUNK, ROW), jnp.float32),
            pltpu.SemaphoreType.DMA,
        ],
        compiler_params=pltpu.CompilerParams(use_tc_tiling_on_sc=False),
    )
    def gather_kernel(table_hbm, idx_hbm, out_hbm, idx_v, rows_v, sem):
        wid = lax.axis_index("s") * 2 + lax.axis_index("c")
        base = pl.multiple_of(wid * PER_W, 8)

        def body(j, carry):
            off = pl.multiple_of(base + j * CHUNK, 8)
            pltpu.sync_copy(idx_hbm.at[pl.ds(off, CHUNK)], idx_v)
            pltpu.async_copy(table_hbm.at[idx_v], rows_v, sem).wait()
            pltpu.sync_copy(rows_v, out_hbm.at[pl.ds(off, CHUNK)])
            return carry

        lax.fori_loop(0, NCHUNK, body, 0)

    return gather_kernel(table, idx_flat)


def _tc_body(xyz_ref, g_ref,
             d_ref, t_ref, w1_ref, b1_ref, w2_ref, b2_ref, w3_ref, b3_ref,
             e_ref, f_ref, lwt_ref, lb_ref,
             local_ref, out_ref):
    x = xyz_ref[...]
    g = g_ref[...]
    f32 = jnp.float32
    local = (jnp.dot(g, d_ref[...], preferred_element_type=f32)
             - jnp.dot(x, t_ref[...], preferred_element_type=f32))
    local_ref[...] = local

    h1 = jnp.maximum(jnp.dot(local, w1_ref[...], preferred_element_type=f32)
                     + b1_ref[...], 0.0)
    h2 = jnp.dot(h1, w2_ref[...], preferred_element_type=f32) + b2_ref[...]
    w = jnp.maximum(jnp.dot(h2, w3_ref[...], preferred_element_type=f32)
                    + b3_ref[...], 0.0)

    e = e_ref[...]
    f = f_ref[...]
    pre = jnp.zeros((B, C_IN * C_MID), f32)
    for k in range(K):
        gfk = g[:, k * ROW:k * ROW + C_IN]
        wk = w[:, k * C_MID:(k + 1) * C_MID]
        pre = pre + (jnp.dot(gfk, e, preferred_element_type=f32)
                     * jnp.dot(wk, f, preferred_element_type=f32))

    out = jnp.dot(pre, lwt_ref[...], preferred_element_type=f32) + lb_ref[...]
    out_ref[...] = jnp.maximum(out, 0.0)


def _tc_dense(xyz, g, d, t, w1b, b1b, w2b, b2b, w3b, b3b, e, f, lwt, lb2):
    full = lambda shape: pl.BlockSpec(shape, lambda i: (0, 0))
    row = lambda width: pl.BlockSpec((B, width), lambda i: (i, 0))
    return pl.pallas_call(
        _tc_body,
        grid=(GRID,),
        in_specs=[
            row(3), row(ROW * K),
            full((ROW * K, 3 * K)), full((3, 3 * K)),
            full((3 * K, 8 * K)), full((1, 8 * K)),
            full((8 * K, 8 * K)), full((1, 8 * K)),
            full((8 * K, C_MID * K)), full((1, C_MID * K)),
            full((C_IN, C_IN * C_MID)), full((C_MID, C_IN * C_MID)),
            full((C_IN * C_MID, C_OUT)), full((1, C_OUT)),
        ],
        out_specs=[
            pl.BlockSpec((B, 3 * K), lambda i: (i, 0)),
            pl.BlockSpec((B, C_OUT), lambda i: (i, 0)),
        ],
        out_shape=[
            jax.ShapeDtypeStruct((N, 3 * K), jnp.float32),
            jax.ShapeDtypeStruct((N, C_OUT), jnp.float32),
        ],
    )(xyz, g, d, t, w1b, b1b, w2b, b2b, w3b, b3b, e, f, lwt, lb2)


def kernel(dense_xyz, dense_feats, nei_inds,
           w1, b1, g1, be1, w2, b2, g2, be2, w3, b3, g3, be3, lw, lb):
    xyz = dense_xyz[0]
    feats = dense_feats[0]
    idx_flat = nei_inds[0].reshape(-1).astype(jnp.int32)
    table = jnp.concatenate(
        [feats, xyz, jnp.zeros((N, ROW - C_IN - 3), jnp.float32)], axis=1)

    grows = _sc_gather(table, idx_flat)
    g2d = grows.reshape(N, K * ROW)

    inv = 1.0 / jnp.sqrt(1.0 + 1e-5)
    s1, s2, s3 = g1 * inv, g2 * inv, g3 * inv
    w1e = w1.T * s1[None, :]
    c1 = b1 * s1 + be1
    w2e = w2.T * s2[None, :]
    c2 = b2 * s2 + be2
    w3e = w3.T * s3[None, :]
    c3 = b3 * s3 + be3

    eyeK = jnp.eye(K, dtype=jnp.float32)
    w1b = jnp.kron(eyeK, w1e)
    w2b = jnp.kron(eyeK, w2e)
    w3b = jnp.kron(eyeK, w3e)
    b1b = jnp.tile(c1, K)[None, :]
    b2b = jnp.tile(c2, K)[None, :]
    b3b = jnp.tile(c3, K)[None, :]

    dk = jnp.zeros((ROW, 3), jnp.float32).at[C_IN:C_IN + 3, :].set(
        jnp.eye(3, dtype=jnp.float32))
    d = jnp.kron(eyeK, dk)
    t = jnp.tile(jnp.eye(3, dtype=jnp.float32), (1, K))
    e = jnp.kron(jnp.eye(C_IN, dtype=jnp.float32),
                 jnp.ones((1, C_MID), jnp.float32))
    f = jnp.tile(jnp.eye(C_MID, dtype=jnp.float32), (1, C_IN))

    local48, new_feat = _tc_dense(xyz, g2d, d, t,
                                  w1b, b1b, w2b, b2b, w3b, b3b,
                                  e, f, lw.T, lb[None, :])
    return (new_feat[None], local48.reshape(1, N, K, 3))
